# baseline (speedup 1.0000x reference)
import functools

import jax
import jax.numpy as jnp
from jax import lax
from jax.experimental import pallas as pl
from jax.experimental.pallas import tpu as pltpu
from jax.experimental.pallas import tpu_sc as plsc

N_NODES = 10000
D = 128
N_EDGES = 320000
CHUNK = 128
NC = 2
NS = 16
NW = NC * NS
EPW = 80
N_CHUNKS_PAD = NW * EPW
N_EDGES_PAD = N_CHUNKS_PAD * CHUNK
N_PAD = 10240
RPT = N_PAD // NS
N_PAD2 = 10112
RPT2 = N_PAD2 // NS
BLK = 1000

_MESH = plsc.VectorSubcoreMesh(core_axis_name="c", subcore_axis_name="s")


@functools.partial(
    pl.kernel,
    out_type=jax.ShapeDtypeStruct((NC * N_PAD,), jnp.float32),
    mesh=_MESH,
    scratch_types=[
        pltpu.VMEM((EPW, CHUNK), jnp.int32),
        pltpu.VMEM((CHUNK,), jnp.float32),
        pltpu.VMEM((CHUNK,), jnp.float32),
        pltpu.SemaphoreType.DMA,
        pltpu.VMEM_SHARED((N_PAD,), jnp.float32),
    ],
)
def _deg_kernel(dst_hbm, ones_hbm, zeros_hbm, deg_hbm, dst_v, ones_v, zeros_v, dsem, deg_sp):
    cid = lax.axis_index("c")
    sid = lax.axis_index("s")
    wid = sid * NC + cid
    pltpu.sync_copy(ones_hbm, ones_v)
    pltpu.sync_copy(zeros_hbm, zeros_v)
    for k in range(RPT // CHUNK):
        base = pl.multiple_of(sid * RPT + k * CHUNK, CHUNK)
        pltpu.sync_copy(zeros_v, deg_sp.at[pl.ds(base, CHUNK)])
    pltpu.sync_copy(dst_hbm.at[pl.ds(wid * EPW, EPW)], dst_v)
    plsc.subcore_barrier()

    @pl.loop(0, EPW // 8)
    def _(g):
        c0 = g * 8
        for q in range(8):
            pltpu.async_copy(ones_v, deg_sp.at[dst_v.at[c0 + q]], dsem, add=True)
        for q in range(8):
            pltpu.make_async_copy(ones_v, deg_sp.at[dst_v.at[0]], dsem).wait()

    plsc.subcore_barrier()
    src_base = pl.multiple_of(sid * RPT, 8)
    dst_base = pl.multiple_of(cid * N_PAD + sid * RPT, 8)
    pltpu.sync_copy(deg_sp.at[pl.ds(src_base, RPT)], deg_hbm.at[pl.ds(dst_base, RPT)])


@functools.partial(
    pl.kernel,
    out_type=jax.ShapeDtypeStruct((NC * N_PAD2, D), jnp.float32),
    mesh=_MESH,
    scratch_types=[
        pltpu.VMEM((3, CHUNK), jnp.int32),
        pltpu.VMEM((3, CHUNK), jnp.int32),
        pltpu.VMEM((3, CHUNK, D), jnp.float32),
        pltpu.SemaphoreType.DMA,
        pltpu.SemaphoreType.DMA,
        pltpu.SemaphoreType.DMA,
        pltpu.SemaphoreType.DMA,
        pltpu.SemaphoreType.DMA,
        pltpu.VMEM_SHARED((N_PAD2, D), jnp.float32),
    ],
)
def _scatter_kernel(y_hbm, src_hbm, dst_hbm, zeros_hbm, out_hbm,
                    src_i, dst_i, rows_v, isem, gsem, ss0, ss1, ss2, acc_sp):
    ssem = (ss0, ss1, ss2)
    cid = lax.axis_index("c")
    sid = lax.axis_index("s")
    wid = sid * NC + cid
    cbase = wid * EPW

    def load_idx(c, t):
        a = pltpu.async_copy(src_hbm.at[cbase + c], src_i.at[t], isem)
        b = pltpu.async_copy(dst_hbm.at[cbase + c], dst_i.at[t], isem)
        return (a, b)

    def gather(t):
        return pltpu.async_copy(y_hbm.at[src_i.at[t]], rows_v.at[t], gsem)

    def scatter(t):
        return pltpu.async_copy(rows_v.at[t], acc_sp.at[dst_i.at[t]], ssem[t],
                                add=True)

    def wait_scatter(t):
        pltpu.make_async_copy(rows_v.at[t], acc_sp.at[dst_i.at[0]],
                              ssem[t]).wait()

    pltpu.sync_copy(zeros_hbm, rows_v.at[0])
    for k in range(4):
        base = pl.multiple_of((sid + NS * k) * CHUNK, CHUNK)
        pltpu.sync_copy(rows_v.at[0], acc_sp.at[pl.ds(base, CHUNK)])

    @pl.when(sid < NS - 1)
    def _():
        base = pl.multiple_of((sid + NS * 4) * CHUNK, CHUNK)
        pltpu.sync_copy(rows_v.at[0], acc_sp.at[pl.ds(base, CHUNK)])

    plsc.subcore_barrier()

    i0 = load_idx(0, 0)
    i1 = load_idx(1, 1)
    for d_ in i0:
        d_.wait()
    gather(0).wait()
    for d_ in i1:
        d_.wait()

    def body(c, ai, q, first=False):
        gg = gather((q + 1) % 3)
        if not first:
            wait_scatter((q + 2) % 3)
        i2 = load_idx(ai, (q + 2) % 3)
        scatter(q)
        for d_ in i2:
            d_.wait()
        gg.wait()

    body(0, 2, 0, first=True)
    body(1, 3, 1)

    @pl.loop(0, (EPW - 2) // 3)
    def _(p):
        c0 = p * 3 + 2
        for q_ in range(3):
            c = c0 + q_
            a = c + 2
            ai = jnp.where(a >= EPW, a - EPW, a)
            body(c, ai, (2 + q_) % 3)

    wait_scatter(1)
    plsc.subcore_barrier()
    pltpu.sync_copy(acc_sp.at[pl.ds(sid * RPT2, RPT2)],
                    out_hbm.at[pl.ds(cid * N_PAD2 + sid * RPT2, RPT2)])


def _mm1_body(x_ref, w_ref, deg_ref, y_ref, dinv_ref):
    deg = deg_ref[0] + deg_ref[1] + 1.0
    dinv = lax.rsqrt(deg)
    dinv_ref[...] = dinv
    y_ref[...] = jnp.dot(x_ref[...], w_ref[...],
                         preferred_element_type=jnp.float32) * dinv


_mm1_call = pl.pallas_call(
    _mm1_body,
    grid=(N_NODES // BLK,),
    in_specs=[
        pl.BlockSpec((BLK, D), lambda i: (i, 0)),
        pl.BlockSpec((D, D), lambda i: (0, 0)),
        pl.BlockSpec((2, BLK, 1), lambda i: (0, i, 0)),
    ],
    out_specs=[
        pl.BlockSpec((BLK, D), lambda i: (i, 0)),
        pl.BlockSpec((BLK, 1), lambda i: (i, 0)),
    ],
    out_shape=[
        jax.ShapeDtypeStruct((N_NODES, D), jnp.float32),
        jax.ShapeDtypeStruct((N_NODES, 1), jnp.float32),
    ],
)


def _layer_body(agg_ref, y1_ref, dinv_ref, b_ref, p_ref, pm_ref, w_ref, y2_ref):
    dinv = dinv_ref[...]
    a = (agg_ref[0] + agg_ref[1] + y1_ref[...]) * dinv
    h = jnp.maximum(a + b_ref[...] + p_ref[...] * pm_ref[0, 0], 0.0)
    y2_ref[...] = jnp.dot(h, w_ref[...],
                          preferred_element_type=jnp.float32) * dinv


_layer_call = pl.pallas_call(
    _layer_body,
    grid=(N_NODES // BLK,),
    in_specs=[
        pl.BlockSpec((2, BLK, D), lambda i: (0, i, 0)),
        pl.BlockSpec((BLK, D), lambda i: (i, 0)),
        pl.BlockSpec((BLK, 1), lambda i: (i, 0)),
        pl.BlockSpec((1, D), lambda i: (0, 0)),
        pl.BlockSpec((BLK, D), lambda i: (i, 0)),
        pl.BlockSpec(memory_space=pltpu.SMEM),
        pl.BlockSpec((D, D), lambda i: (0, 0)),
    ],
    out_specs=pl.BlockSpec((BLK, D), lambda i: (i, 0)),
    out_shape=jax.ShapeDtypeStruct((N_NODES, D), jnp.float32),
)


def _final_body(agg_ref, y2_ref, dinv_ref, b_ref, p_ref, pm_ref, o_ref):
    a = (agg_ref[0] + agg_ref[1] + y2_ref[...]) * dinv_ref[...]
    o_ref[...] = a + b_ref[...] + p_ref[...] * pm_ref[0, 0]


_final_call = pl.pallas_call(
    _final_body,
    grid=(N_NODES // BLK,),
    in_specs=[
        pl.BlockSpec((2, BLK, D), lambda i: (0, i, 0)),
        pl.BlockSpec((BLK, D), lambda i: (i, 0)),
        pl.BlockSpec((BLK, 1), lambda i: (i, 0)),
        pl.BlockSpec((1, D), lambda i: (0, 0)),
        pl.BlockSpec((BLK, D), lambda i: (i, 0)),
        pl.BlockSpec(memory_space=pltpu.SMEM),
    ],
    out_specs=pl.BlockSpec((BLK, D), lambda i: (i, 0)),
    out_shape=jax.ShapeDtypeStruct((N_NODES, D), jnp.float32),
)


def kernel(x, edge_index, W1, b1, W2, b2, P1, P2, cache_name, perturb):
    ei = edge_index.astype(jnp.int32)
    npad = N_EDGES_PAD - N_EDGES
    seq = jnp.arange(npad, dtype=jnp.int32)
    src2 = jnp.concatenate([ei[0], seq % N_NODES])
    dst2 = jnp.concatenate([ei[1], N_NODES + seq % (N_PAD2 - N_NODES)])
    src2 = src2.reshape(N_CHUNKS_PAD, CHUNK)
    dst2 = dst2.reshape(N_CHUNKS_PAD, CHUNK)
    ones_vec = jnp.ones((CHUNK,), jnp.float32)
    zeros_vec = jnp.zeros((CHUNK,), jnp.float32)
    zeros_mat = jnp.zeros((CHUNK, D), jnp.float32)
    pmask = jnp.where(jnp.asarray(perturb) != 0, 1.0, 0.0).astype(jnp.float32)
    pmask = pmask.reshape(1, 1)

    deg3 = _deg_kernel(dst2, ones_vec, zeros_vec).reshape(NC, N_PAD, 1)
    y1, dinv = _mm1_call(x, W1, deg3)
    agg1 = _scatter_kernel(y1, src2, dst2, zeros_mat).reshape(NC, N_PAD2, D)
    y2 = _layer_call(agg1, y1, dinv, b1.reshape(1, D), P1, pmask, W2)
    agg2 = _scatter_kernel(y2, src2, dst2, zeros_mat).reshape(NC, N_PAD2, D)
    return _final_call(agg2, y2, dinv, b2.reshape(1, D), P2, pmask)
